# baseline (speedup 1.0000x reference)
import functools
import math

import jax
import jax.numpy as jnp
from jax import lax
from jax.experimental import pallas as pl
from jax.experimental.pallas import tpu as pltpu
from jax.experimental.pallas import tpu_sc as plsc

NROW, NCOL = 4096, 200
D = 64
LANES = 16
NC, NS = 2, 16
NW = NC * NS
RPW = NROW // NW
NR = 4
CH = NR * NCOL
G = RPW // NR
REM = NCOL - 128
SCALE = math.sqrt(D)

_mesh = plsc.VectorSubcoreMesh(core_axis_name="c", subcore_axis_name="s")


@functools.partial(
    pl.kernel,
    mesh=_mesh,
    out_type=jax.ShapeDtypeStruct((NROW, NCOL, D), jnp.float32),
    scratch_types=[
        pltpu.VMEM((2, NR, 128), jnp.int32),
        pltpu.VMEM((2, NR, 128), jnp.int32),
        pltpu.VMEM((2, NR, NCOL, D), jnp.float32),
        pltpu.SemaphoreType.DMA,
        pltpu.SemaphoreType.DMA,
        pltpu.SemaphoreType.DMA,
        pltpu.SemaphoreType.DMA,
    ],
    compiler_params=pltpu.CompilerParams(use_tc_tiling_on_sc=False),
)
def _emb_lookup(xa_hbm, xb_hbm, table_hbm, out_hbm, idxa, idxb, rows_v,
                g0, g1, o0, o1):
    wid = lax.axis_index("s") * NC + lax.axis_index("c")
    row_base = wid * RPW
    gsem = (g0, g1)
    osem = (o0, o1)

    def stage(g, b):
        r0 = row_base + g * NR
        pltpu.sync_copy(xa_hbm.at[pl.ds(r0, NR)], idxa.at[b])
        pltpu.sync_copy(xb_hbm.at[pl.ds(r0, NR)], idxb.at[b])
        for r in range(NR):
            pltpu.async_copy(
                table_hbm.at[idxa.at[b, r]],
                rows_v.at[b, r, pl.ds(0, 128)],
                gsem[b],
            )
            pltpu.async_copy(
                table_hbm.at[idxb.at[b, r, pl.ds(0, REM)]],
                rows_v.at[b, r, pl.ds(128, REM)],
                gsem[b],
            )

    def wait_gathers(b):
        for r in range(NR):
            pltpu.make_async_copy(
                table_hbm.at[idxa.at[b, r]],
                rows_v.at[b, r, pl.ds(0, 128)],
                gsem[b],
            ).wait()
            pltpu.make_async_copy(
                table_hbm.at[idxb.at[b, r, pl.ds(0, REM)]],
                rows_v.at[b, r, pl.ds(128, REM)],
                gsem[b],
            ).wait()

    def wait_writeback(b):
        pltpu.make_async_copy(
            rows_v.at[b], out_hbm.at[pl.ds(0, NR)], osem[b]
        ).wait()

    stage(0, 0)

    def pair_body(k, carry):
        for b in (0, 1):
            gc = 2 * k + b
            nxt = gc + 1

            @pl.when(nxt < G)
            def _():
                @pl.when(nxt >= 2)
                def _():
                    wait_writeback(1 - b)

                stage(nxt, 1 - b)

            wait_gathers(b)

            for r in range(NR):
                @plsc.parallel_loop(0, NCOL, 1, unroll=8)
                def _(c):
                    for kk in range(D // LANES):
                        sl = pl.ds(kk * LANES, LANES)
                        rows_v[b, r, c, sl] = rows_v[b, r, c, sl] * SCALE

            pltpu.async_copy(
                rows_v.at[b],
                out_hbm.at[pl.ds(row_base + gc * NR, NR)],
                osem[b],
            )
        return carry

    lax.fori_loop(0, G // 2, pair_body, 0)
    wait_writeback(0)
    wait_writeback(1)


def kernel(x, table):
    xi = x.astype(jnp.int32)
    xa = xi[:, :128]
    xb = jnp.pad(xi[:, 128:], ((0, 0), (0, 128 - REM)))
    return _emb_lookup(xa, xb, table)
